# baseline (speedup 1.0000x reference)
import dataclasses

import jax
import jax.numpy as jnp
import numpy as np
from jax import lax
from jax.experimental import pallas as pl
from jax.experimental.pallas import tpu as pltpu
from jax.experimental.pallas import tpu_sc as plsc

B = 4096
V = 100000
D = 64
F = 26
NF = F + 1
NUM_IDX = B * F
NW = 32
SAMPLES_PER_W = B // NW
IDX_PER_W = NUM_IDX // NW
BS = 512


_NL = 16


def _sc_gather(flat_tables, flat_idx, lane_ids):
    mesh = plsc.VectorSubcoreMesh(core_axis_name="core", subcore_axis_name="subcore")
    n_groups = IDX_PER_W // _NL
    cp = pltpu.CompilerParams()
    if "needs_layout_passes" in pltpu.CompilerParams.__dataclass_fields__:
        cp = dataclasses.replace(cp, needs_layout_passes=False)

    @pl.kernel(
        out_type=jax.ShapeDtypeStruct((NUM_IDX, D), flat_tables.dtype),
        mesh=mesh,
        compiler_params=cp,
        scratch_types=[
            pltpu.VMEM((IDX_PER_W,), jnp.int32),
            pltpu.VMEM((_NL,), jnp.int32),
            pltpu.SemaphoreType.DMA,
        ],
    )
    def gather_kernel(x_hbm, i_hbm, l_hbm, o_hbm, idx_v, lane_v, sem):
        wid = lax.axis_index("subcore") * 2 + lax.axis_index("core")
        base = wid * IDX_PER_W
        pltpu.sync_copy(i_hbm.at[0, pl.ds(base, IDX_PER_W)], idx_v)
        pltpu.sync_copy(l_hbm.at[0], lane_v)
        lanes = lane_v[...]

        @pl.loop(0, n_groups)
        def _(g):
            idx16 = idx_v[pl.ds(g * _NL, _NL)]
            for l in range(_NL):
                idx = jnp.max(jnp.where(lanes == l, idx16, 0))
                pltpu.async_copy(
                    x_hbm.at[pl.ds(idx, 1)],
                    o_hbm.at[pl.ds(base + g * _NL + l, 1)],
                    sem,
                )

        @pl.loop(0, IDX_PER_W)
        def _(j):
            pltpu.make_async_copy(
                x_hbm.at[pl.ds(0, 1)],
                o_hbm.at[pl.ds(base, 1)],
                sem,
            ).wait()

    return gather_kernel(flat_tables, flat_idx, lane_ids)


def _dense_body(x_ref, emb_ref, dw1_, db1_, dw2_, db2_, dw3_, db3_,
                ow1d_, ow1z_, ob1_, ow2_, ob2_, ow3_, ob3_, o_ref):
    f32 = jnp.float32
    x = x_ref[...]
    d = jnp.maximum(jax.lax.dot(x, dw1_[...], preferred_element_type=f32) + db1_[...], 0.0)
    d = jnp.maximum(jax.lax.dot(d, dw2_[...], preferred_element_type=f32) + db2_[...], 0.0)
    d = jnp.maximum(jax.lax.dot(d, dw3_[...], preferred_element_type=f32) + db3_[...], 0.0)
    emb = emb_ref[...].reshape(BS, F, D)
    a = jnp.concatenate([d[:, None, :], emb], axis=1)
    z = jax.lax.dot_general(
        a, a, (((2,), (2,)), ((0,), (0,))), preferred_element_type=f32
    )
    zf = z.reshape(BS, NF * NF)
    h = (jax.lax.dot(d, ow1d_[...], preferred_element_type=f32)
         + jax.lax.dot(zf, ow1z_[...], preferred_element_type=f32)
         + ob1_[...])
    h = jnp.maximum(h, 0.0)
    h = jnp.maximum(jax.lax.dot(h, ow2_[...], preferred_element_type=f32) + ob2_[...], 0.0)
    o_ref[...] = jax.lax.dot(h, ow3_[...], preferred_element_type=f32) + ob3_[...]


_LI, _LJ = np.triu_indices(NF, k=1)
_PAIR_POS = np.asarray(_LI * NF + _LJ)


def kernel(dense_features, sparse_indices, tables, dw1, db1, dw2, db2, dw3,
           db3, ow1, ob1, ow2, ob2, ow3, ob3):
    flat_tables = tables.reshape(F * V, D)
    offs = (jnp.arange(F, dtype=jnp.int32) * V)[None, :]
    flat_idx = (sparse_indices.astype(jnp.int32) + offs).reshape(1, NUM_IDX)
    lane_ids = jnp.arange(_NL, dtype=jnp.int32)[None, :]
    emb2 = _sc_gather(flat_tables, flat_idx, lane_ids).reshape(B, F * D)

    ow1d = ow1[:D]
    ow1z = jnp.zeros((NF * NF, ow1.shape[1]), ow1.dtype).at[_PAIR_POS].set(ow1[D:])

    n_blocks = B // BS
    wspec = lambda shape: pl.BlockSpec(shape, lambda i: (0,) * len(shape))
    out = pl.pallas_call(
        _dense_body,
        grid=(n_blocks,),
        in_specs=[
            pl.BlockSpec((BS, dense_features.shape[1]), lambda i: (i, 0)),
            pl.BlockSpec((BS, F * D), lambda i: (i, 0)),
            wspec(dw1.shape), wspec((1, db1.shape[0])),
            wspec(dw2.shape), wspec((1, db2.shape[0])),
            wspec(dw3.shape), wspec((1, db3.shape[0])),
            wspec(ow1d.shape), wspec(ow1z.shape), wspec((1, ob1.shape[0])),
            wspec(ow2.shape), wspec((1, ob2.shape[0])),
            wspec(ow3.shape), wspec((1, ob3.shape[0])),
        ],
        out_specs=pl.BlockSpec((BS, 1), lambda i: (i, 0)),
        out_shape=jax.ShapeDtypeStruct((B, 1), jnp.float32),
    )(
        dense_features, emb2, dw1, db1[None], dw2, db2[None], dw3,
        db3[None], ow1d, ow1z, ob1[None], ow2, ob2[None], ow3, ob3[None],
    )
    return out
